# baseline (device time: 150818 ns/iter reference)
import jax
import jax.numpy as jnp
from jax import lax
from jax.experimental import pallas as pl
from jax.experimental.pallas import tpu as pltpu

N_DEV = 4
N_PC = 4
N_PC_DIAG = 8


def _body(x_hbm_ref, w_hbm_ref, scale_ref, out_hbm_ref,
          x_ref, w_ref, from_l_ref, from_r_ref, from_o_ref, stage_ref,
          send_sems, recv_sems, x_sem, w_sems, out_sems):
    my_pos = lax.axis_index("i")
    left = lax.rem(my_pos + (N_DEV - 1), N_DEV)
    right = lax.rem(my_pos + 1, N_DEV)
    opp = lax.rem(my_pos + 2, N_DEV)
    m_per = x_ref.shape[0]
    n_per = w_ref.shape[1]
    n_half = n_per // 2

    x_load = pltpu.make_async_copy(x_hbm_ref, x_ref, x_sem)
    x_load.start()
    w_loads = []
    for j in range(2):
        wl = pltpu.make_async_copy(
            w_hbm_ref.at[:, pl.ds(my_pos * n_per + j * n_half, n_half)],
            w_ref.at[:, pl.ds(j * n_half, n_half)],
            w_sems.at[j],
        )
        wl.start()
        w_loads.append(wl)

    barrier_sem = pltpu.get_barrier_semaphore()
    for nbr in (left, right, opp):
        pl.semaphore_signal(
            barrier_sem, inc=1,
            device_id=(nbr,), device_id_type=pl.DeviceIdType.MESH,
        )
    pl.semaphore_wait(barrier_sem, 3)

    def peer_rdmas(peer, peer_idx, dst_ref, n_pieces):
        m_q = m_per // n_pieces
        rs = []
        for q in range(n_pieces):
            rows = pl.ds(q * m_q, m_q)
            rs.append(pltpu.make_async_remote_copy(
                src_ref=x_hbm_ref.at[rows, :],
                dst_ref=dst_ref.at[rows, :],
                send_sem=send_sems.at[peer_idx, q],
                recv_sem=recv_sems.at[peer_idx, q],
                device_id=(peer,), device_id_type=pl.DeviceIdType.MESH,
            ))
        return rs

    to_o = peer_rdmas(opp, 2, from_o_ref, N_PC_DIAG)
    to_l = peer_rdmas(left, 0, from_r_ref, N_PC)
    to_r = peer_rdmas(right, 1, from_l_ref, N_PC)

    for r in to_o + to_l + to_r:
        r.start()

    scale = scale_ref[0]

    out_copies = [None, None]
    slot_cycle = [0]

    def store(block, origin_row_start, rows):
        slot = slot_cycle[0] % 2
        slot_cycle[0] += 1
        if out_copies[slot] is not None:
            out_copies[slot].wait()
        stage_ref[slot, pl.ds(0, rows)] = block
        cp = pltpu.make_async_copy(
            stage_ref.at[slot, pl.ds(0, rows)],
            out_hbm_ref.at[pl.ds(origin_row_start, rows), :],
            out_sems.at[slot],
        )
        cp.start()
        out_copies[slot] = cp

    def matmul(chunk, w_block):
        return lax.dot_general(
            chunk, w_block,
            (((1,), (0,)), ((), ())),
            preferred_element_type=jnp.int32,
        )

    x_load.wait()
    w_loads[0].wait()
    own0 = matmul(x_ref[...], w_ref[:, pl.ds(0, n_half)])
    stage_ref[0, :, pl.ds(0, n_half)] = own0.astype(jnp.float32) * scale
    w_loads[1].wait()
    own1 = matmul(x_ref[...], w_ref[:, pl.ds(n_half, n_half)])
    stage_ref[0, :, pl.ds(n_half, n_half)] = own1.astype(jnp.float32) * scale
    cp = pltpu.make_async_copy(
        stage_ref.at[0],
        out_hbm_ref.at[pl.ds(my_pos * m_per, m_per), :],
        out_sems.at[0],
    )
    cp.start()
    out_copies[0] = cp
    slot_cycle[0] = 1

    def consume(rdma, src_ref, origin, q, n_pieces):
        m_q = m_per // n_pieces
        rdma.wait_recv()
        rows = pl.ds(q * m_q, m_q)
        piece = matmul(src_ref[rows, :], w_ref[...]).astype(jnp.float32) * scale
        store(piece, origin * m_per + q * m_q, m_q)

    for q in range(N_PC):
        consume(to_r[q], from_l_ref, left, q, N_PC)
        consume(to_l[q], from_r_ref, right, q, N_PC)
    for q in range(N_PC_DIAG):
        consume(to_o[q], from_o_ref, opp, q, N_PC_DIAG)

    for r in to_o + to_l + to_r:
        r.wait_send()
    out_copies[0].wait()
    out_copies[1].wait()


def kernel(x, w_mat, scale_x, scale_w):
    m_per, k = x.shape
    n = w_mat.shape[1]
    n_per = n // N_DEV

    scale = (scale_x * scale_w).astype(jnp.float32)

    return pl.pallas_call(
        _body,
        out_shape=jax.ShapeDtypeStruct((N_DEV * m_per, n_per), jnp.float32),
        in_specs=[
            pl.BlockSpec(memory_space=pl.ANY),
            pl.BlockSpec(memory_space=pl.ANY),
            pl.BlockSpec(memory_space=pltpu.SMEM),
        ],
        out_specs=pl.BlockSpec(memory_space=pl.ANY),
        scratch_shapes=[
            pltpu.VMEM((m_per, k), x.dtype),
            pltpu.VMEM((k, n_per), w_mat.dtype),
            pltpu.VMEM((m_per, k), x.dtype),
            pltpu.VMEM((m_per, k), x.dtype),
            pltpu.VMEM((m_per, k), x.dtype),
            pltpu.VMEM((2, m_per, n_per), jnp.float32),
            pltpu.SemaphoreType.DMA((3, N_PC_DIAG)),
            pltpu.SemaphoreType.DMA((3, N_PC_DIAG)),
            pltpu.SemaphoreType.DMA,
            pltpu.SemaphoreType.DMA((2,)),
            pltpu.SemaphoreType.DMA((2,)),
        ],
        compiler_params=pltpu.CompilerParams(
            collective_id=0,
            vmem_limit_bytes=100 * 1024 * 1024,
        ),
    )(x, w_mat, scale)


# device time: 150053 ns/iter; 1.0051x vs baseline; 1.0051x over previous
import jax
import jax.numpy as jnp
from jax import lax
from jax.experimental import pallas as pl
from jax.experimental.pallas import tpu as pltpu

N_DEV = 4
N_PIECES = 4


def _body(x_ref, w_hbm_ref, scale_ref, out_hbm_ref,
          w_ref, from_l_ref, from_r_ref, from_o_ref, stage_ref,
          send_sems, recv_sems, w_sems, out_sems):
    my_pos = lax.axis_index("i")
    left = lax.rem(my_pos + (N_DEV - 1), N_DEV)
    right = lax.rem(my_pos + 1, N_DEV)
    opp = lax.rem(my_pos + 2, N_DEV)
    m_per = x_ref.shape[0]
    n_per = w_ref.shape[1]
    m_q = m_per // N_PIECES
    n_half = n_per // 2

    w_loads = []
    for j in range(2):
        wl = pltpu.make_async_copy(
            w_hbm_ref.at[:, pl.ds(my_pos * n_per + j * n_half, n_half)],
            w_ref.at[:, pl.ds(j * n_half, n_half)],
            w_sems.at[j],
        )
        wl.start()
        w_loads.append(wl)

    barrier_sem = pltpu.get_barrier_semaphore()
    for nbr in (left, right, opp):
        pl.semaphore_signal(
            barrier_sem, inc=1,
            device_id=(nbr,), device_id_type=pl.DeviceIdType.MESH,
        )
    pl.semaphore_wait(barrier_sem, 3)

    def peer_rdmas(peer, peer_sems_idx, dst_ref):
        rs = []
        for q in range(N_PIECES):
            rows = pl.ds(q * m_q, m_q)
            rs.append(pltpu.make_async_remote_copy(
                src_ref=x_ref.at[rows, :],
                dst_ref=dst_ref.at[rows, :],
                send_sem=send_sems.at[peer_sems_idx, q],
                recv_sem=recv_sems.at[peer_sems_idx, q],
                device_id=(peer,), device_id_type=pl.DeviceIdType.MESH,
            ))
        return rs

    to_o = peer_rdmas(opp, 2, from_o_ref)
    to_l = peer_rdmas(left, 0, from_r_ref)
    to_r = peer_rdmas(right, 1, from_l_ref)

    for r in to_o + to_l + to_r:
        r.start()

    scale = scale_ref[0]

    out_copies = [None, None]
    slot_cycle = [0]

    def store(block, origin_row_start, rows):
        slot = slot_cycle[0] % 2
        slot_cycle[0] += 1
        if out_copies[slot] is not None:
            out_copies[slot].wait()
        stage_ref[slot, pl.ds(0, rows)] = block
        cp = pltpu.make_async_copy(
            stage_ref.at[slot, pl.ds(0, rows)],
            out_hbm_ref.at[pl.ds(origin_row_start, rows), :],
            out_sems.at[slot],
        )
        cp.start()
        out_copies[slot] = cp

    def matmul(chunk, w_block):
        return lax.dot_general(
            chunk, w_block,
            (((1,), (0,)), ((), ())),
            preferred_element_type=jnp.int32,
        )

    w_loads[0].wait()
    own0 = matmul(x_ref[...], w_ref[:, pl.ds(0, n_half)])
    stage_ref[0, :, pl.ds(0, n_half)] = own0.astype(jnp.float32) * scale
    w_loads[1].wait()
    own1 = matmul(x_ref[...], w_ref[:, pl.ds(n_half, n_half)])
    stage_ref[0, :, pl.ds(n_half, n_half)] = own1.astype(jnp.float32) * scale
    cp = pltpu.make_async_copy(
        stage_ref.at[0],
        out_hbm_ref.at[pl.ds(my_pos * m_per, m_per), :],
        out_sems.at[0],
    )
    cp.start()
    out_copies[0] = cp
    slot_cycle[0] = 1

    def consume(rdma, src_ref, origin, q):
        rdma.wait_recv()
        rows = pl.ds(q * m_q, m_q)
        piece = matmul(src_ref[rows, :], w_ref[...]).astype(jnp.float32) * scale
        store(piece, origin * m_per + q * m_q, m_q)

    for q in range(N_PIECES):
        consume(to_r[q], from_l_ref, left, q)
        consume(to_l[q], from_r_ref, right, q)
    for q in range(N_PIECES):
        consume(to_o[q], from_o_ref, opp, q)

    for r in to_o + to_l + to_r:
        r.wait_send()
    out_copies[0].wait()
    out_copies[1].wait()


def kernel(x, w_mat, scale_x, scale_w):
    m_per, k = x.shape
    n = w_mat.shape[1]
    n_per = n // N_DEV

    scale = (scale_x * scale_w).astype(jnp.float32)

    return pl.pallas_call(
        _body,
        out_shape=jax.ShapeDtypeStruct((N_DEV * m_per, n_per), jnp.float32),
        in_specs=[
            pl.BlockSpec(memory_space=pltpu.VMEM),
            pl.BlockSpec(memory_space=pl.ANY),
            pl.BlockSpec(memory_space=pltpu.SMEM),
        ],
        out_specs=pl.BlockSpec(memory_space=pl.ANY),
        scratch_shapes=[
            pltpu.VMEM((k, n_per), w_mat.dtype),
            pltpu.VMEM((m_per, k), x.dtype),
            pltpu.VMEM((m_per, k), x.dtype),
            pltpu.VMEM((m_per, k), x.dtype),
            pltpu.VMEM((2, m_per, n_per), jnp.float32),
            pltpu.SemaphoreType.DMA((3, N_PIECES)),
            pltpu.SemaphoreType.DMA((3, N_PIECES)),
            pltpu.SemaphoreType.DMA((2,)),
            pltpu.SemaphoreType.DMA((2,)),
        ],
        compiler_params=pltpu.CompilerParams(
            collective_id=0,
            vmem_limit_bytes=100 * 1024 * 1024,
        ),
    )(x, w_mat, scale)
